# baseline (speedup 1.0000x reference)
import functools

import jax
import jax.numpy as jnp
from jax.experimental import pallas as pl
from jax.experimental.pallas import tpu as pltpu


def _pool_matmul_kernel(x_ref, pw_ref, o_ref):
    o_ref[...] = jnp.dot(
        x_ref[...], pw_ref[...], preferred_element_type=jnp.float32
    ).astype(o_ref.dtype)


def _avg_pool(x, d):
    B, C, H, W = x.shape
    Ho, Wo = H // d, W // d
    if H != Ho * d or W != Wo * d:
        x = x[:, :, : Ho * d, : Wo * d]
        H, W = Ho * d, Wo * d
    M = B * C * Ho
    K = d * W

    a = x.reshape(M, K)

    k = jnp.arange(K)
    pw = ((k % W) // d)[:, None] == jnp.arange(Wo)[None, :]
    pw = pw.astype(jnp.float32) * (1.0 / (d * d))

    tm = 16384
    while M % tm and tm > 8:
        tm //= 2
    grid = (pl.cdiv(M, tm),)

    itemsize = x.dtype.itemsize
    cost = pl.CostEstimate(
        flops=2 * M * K * Wo,
        transcendentals=0,
        bytes_accessed=M * K * itemsize + K * Wo * 4 + M * Wo * itemsize,
    )

    out = pl.pallas_call(
        _pool_matmul_kernel,
        out_shape=jax.ShapeDtypeStruct((M, Wo), x.dtype),
        grid=grid,
        in_specs=[
            pl.BlockSpec((tm, K), lambda i: (i, 0)),
            pl.BlockSpec((K, Wo), lambda i: (0, 0)),
        ],
        out_specs=pl.BlockSpec((tm, Wo), lambda i: (i, 0)),
        compiler_params=pltpu.CompilerParams(
            dimension_semantics=("parallel",),
            vmem_limit_bytes=64 << 20,
        ),
        cost_estimate=cost,
    )(a, pw)

    return out.reshape(B, C, Ho, Wo)


def kernel(x):
    return _avg_pool(x, 2)


# speedup vs baseline: 1.0075x; 1.0075x over previous
import functools

import jax
import jax.numpy as jnp
from jax.experimental import pallas as pl
from jax.experimental.pallas import tpu as pltpu

_S = 4


def _pool_kernel(*refs, tq, nsub):
    pw_ref = refs[_S]
    o_ref = refs[_S + 1]
    for j in range(_S):
        o_ref[j * tq:(j + 1) * tq, :] = jnp.dot(
            refs[j][...], pw_ref[...], preferred_element_type=jnp.float32
        ).astype(o_ref.dtype)


def _avg_pool(x, d):
    B, C, H, W = x.shape
    Ho, Wo = H // d, W // d
    if H != Ho * d or W != Wo * d:
        x = x[:, :, : Ho * d, : Wo * d]
        H, W = Ho * d, Wo * d
    M = B * C * Ho
    K = d * W

    a = x.reshape(M, K)

    k = jnp.arange(K)
    pw = ((k % W) // d)[:, None] == jnp.arange(Wo)[None, :]
    pw = pw.astype(jnp.float32) * (1.0 / (d * d))

    tm = 8192
    while M % tm and tm > 8:
        tm //= 2
    tq = tm // _S
    nb = M // tm
    grid = (nb,)

    itemsize = x.dtype.itemsize
    cost = pl.CostEstimate(
        flops=2 * M * K * Wo,
        transcendentals=0,
        bytes_accessed=M * K * itemsize + K * Wo * 4 + M * Wo * itemsize,
    )

    def in_map(j):
        return lambda i: (i * _S + j, 0)

    out = pl.pallas_call(
        functools.partial(_pool_kernel, tq=tq, nsub=_S),
        out_shape=jax.ShapeDtypeStruct((M, Wo), x.dtype),
        grid=grid,
        in_specs=[pl.BlockSpec((tq, K), in_map(j)) for j in range(_S)]
        + [pl.BlockSpec((K, Wo), lambda i: (0, 0))],
        out_specs=pl.BlockSpec((tm, Wo), lambda i: (i, 0)),
        compiler_params=pltpu.CompilerParams(
            dimension_semantics=("parallel",),
            vmem_limit_bytes=64 << 20,
        ),
        cost_estimate=cost,
    )(*([a] * _S), pw)

    return out.reshape(B, C, Ho, Wo)


def kernel(x):
    return _avg_pool(x, 2)


# speedup vs baseline: 3.1173x; 3.0942x over previous
import functools

import jax
import jax.numpy as jnp
from jax.experimental import pallas as pl
from jax.experimental.pallas import tpu as pltpu


def _pool_kernel(x_ref, pw_ref, o_ref, *, d):
    xv = x_ref[...]
    tr, w = xv.shape
    z = xv.reshape(tr // d, d * w)
    xs = z[:, 0:w].astype(jnp.float32)
    for j in range(1, d):
        xs = xs + z[:, j * w:(j + 1) * w]
    o_ref[...] = jnp.dot(
        xs, pw_ref[...], preferred_element_type=jnp.float32
    ).astype(o_ref.dtype)


def _avg_pool(x, d):
    B, C, H, W = x.shape
    Ho, Wo = H // d, W // d
    if H != Ho * d or W != Wo * d:
        x = x[:, :, : Ho * d, : Wo * d]
        H, W = Ho * d, Wo * d
    R = B * C * H

    a = x.reshape(R, W)

    pw = (jnp.arange(W) // d)[:, None] == jnp.arange(Wo)[None, :]
    pw = pw.astype(jnp.float32) * (1.0 / (d * d))

    tr = 16384
    while R % tr and tr > d * 8:
        tr //= 2
    grid = (R // tr,)

    itemsize = x.dtype.itemsize
    cost = pl.CostEstimate(
        flops=R * W + 2 * (R // d) * W * Wo,
        transcendentals=0,
        bytes_accessed=R * W * itemsize + W * Wo * 4 + (R // d) * Wo * itemsize,
    )

    out = pl.pallas_call(
        functools.partial(_pool_kernel, d=d),
        out_shape=jax.ShapeDtypeStruct((R // d, Wo), x.dtype),
        grid=grid,
        in_specs=[
            pl.BlockSpec((tr, W), lambda i: (i, 0)),
            pl.BlockSpec((W, Wo), lambda i: (0, 0)),
        ],
        out_specs=pl.BlockSpec((tr // d, Wo), lambda i: (i, 0)),
        compiler_params=pltpu.CompilerParams(
            dimension_semantics=("parallel",),
            vmem_limit_bytes=64 << 20,
        ),
        cost_estimate=cost,
    )(a, pw)

    return out.reshape(B, C, Ho, Wo)


def kernel(x):
    return _avg_pool(x, 2)
